# Initial kernel scaffold; baseline (speedup 1.0000x reference)
import jax
import jax.numpy as jnp
from jax.experimental import pallas as pl


def kernel(node_embeddings, node_type_embeddings, trans_weights, trans_weights_s1, trans_weights_s2, train_inputs, train_labels, train_types, node_neigh):
    raise NotImplementedError("write your pallas kernel here")


# speedup vs baseline: 61.4447x; 61.4447x over previous
import functools

import jax
import jax.numpy as jnp
from jax import lax
from jax.experimental import pallas as pl
from jax.experimental.pallas import tpu as pltpu
from jax.experimental.pallas import tpu_sc as plsc

NUM_NODES = 100000
EMB = 64
EMB_U = 16
T = 2
DIM_A = 20
NEIGH = 10
B = 16384

NW = 32
EX_PER_W = B // NW
CH = 128
NCHUNK = EX_PER_W // CH
SLOT = T * NEIGH
IDX_PER_CHUNK = CH * SLOT
GK = 128
NGATHER = IDX_PER_CHUNK // GK


def _sc_body(ne_hbm, ntt_hbm, ti_hbm, nn_hbm,
             out_ne, out_t0, out_t1,
             idx_v, rows_v, acc0_v, acc1_v, tidx_v, nerows_v, sem, sem2):
    c = lax.axis_index("c")
    s = lax.axis_index("s")
    wid = s * 2 + c
    base = wid * EX_PER_W
    for ck in range(NCHUNK):
        g0 = base + ck * CH
        pltpu.sync_copy(nn_hbm.at[pl.ds(g0 * SLOT, IDX_PER_CHUNK)], idx_v)

        def _fix(j, carry):
            pos = j * 16 + lax.iota(jnp.int32, 16)
            t = lax.rem(lax.div(pos, NEIGH), T)
            idx_v[pl.ds(j * 16, 16)] = idx_v[pl.ds(j * 16, 16)] * T + t
            return carry

        lax.fori_loop(0, IDX_PER_CHUNK // 16, _fix, 0)

        descs = [
            pltpu.async_copy(
                ntt_hbm.at[idx_v.at[pl.ds(k * GK, GK)]],
                rows_v.at[pl.ds(k * GK, GK)], sem)
            for k in range(NGATHER)
        ]
        for d in descs:
            d.wait()

        def _accum(e, carry):
            r0 = rows_v[e * SLOT]
            for n in range(1, NEIGH):
                r0 = r0 + rows_v[e * SLOT + n]
            r1 = rows_v[e * SLOT + NEIGH]
            for n in range(NEIGH + 1, SLOT):
                r1 = r1 + rows_v[e * SLOT + n]
            acc0_v[e] = r0 * (1.0 / NEIGH)
            acc1_v[e] = r1 * (1.0 / NEIGH)
            return carry

        lax.fori_loop(0, CH, _accum, 0)
        pltpu.sync_copy(acc0_v, out_t0.at[pl.ds(g0, CH)])
        pltpu.sync_copy(acc1_v, out_t1.at[pl.ds(g0, CH)])

        pltpu.sync_copy(ti_hbm.at[pl.ds(g0, CH)], tidx_v)
        pltpu.async_copy(ne_hbm.at[tidx_v], nerows_v, sem2).wait()
        pltpu.sync_copy(nerows_v, out_ne.at[pl.ds(g0, CH)])


@functools.cache
def _build_sc_gather():
    return pl.kernel(
        _sc_body,
        out_type=[
            jax.ShapeDtypeStruct((B, EMB), jnp.float32),
            jax.ShapeDtypeStruct((B, EMB_U), jnp.float32),
            jax.ShapeDtypeStruct((B, EMB_U), jnp.float32),
        ],
        mesh=plsc.VectorSubcoreMesh(core_axis_name="c", subcore_axis_name="s"),
        compiler_params=pltpu.CompilerParams(use_tc_tiling_on_sc=False),
        scratch_types=[
            pltpu.VMEM((IDX_PER_CHUNK,), jnp.int32),
            pltpu.VMEM((IDX_PER_CHUNK, EMB_U), jnp.float32),
            pltpu.VMEM((CH, EMB_U), jnp.float32),
            pltpu.VMEM((CH, EMB_U), jnp.float32),
            pltpu.VMEM((CH,), jnp.int32),
            pltpu.VMEM((CH, EMB), jnp.float32),
            pltpu.SemaphoreType.DMA,
            pltpu.SemaphoreType.DMA,
        ],
    )


def _tc_body(types_ref, ne_ref, t0_ref, t1_ref, s1_ref, s2_ref, w_ref, out_ref):
    nte0 = t0_ref[...]
    nte1 = t1_ref[...]
    is0 = types_ref[...] == 0
    logit = []
    for tt in range(T):
        s1t = s1_ref[tt]
        s2t = s2_ref[tt]
        h0 = jnp.tanh(jnp.dot(nte0, s1t, preferred_element_type=jnp.float32))
        h1 = jnp.tanh(jnp.dot(nte1, s1t, preferred_element_type=jnp.float32))
        logit.append((jnp.sum(h0 * s2t, axis=1, keepdims=True),
                      jnp.sum(h1 * s2t, axis=1, keepdims=True)))
    l0 = jnp.where(is0, logit[0][0], logit[1][0])
    l1 = jnp.where(is0, logit[0][1], logit[1][1])
    m = jnp.maximum(l0, l1)
    e0 = jnp.exp(l0 - m)
    e1 = jnp.exp(l1 - m)
    inv = 1.0 / (e0 + e1)
    comb = (e0 * inv) * nte0 + (e1 * inv) * nte1
    d0 = jnp.dot(comb, w_ref[0], preferred_element_type=jnp.float32)
    d1 = jnp.dot(comb, w_ref[1], preferred_element_type=jnp.float32)
    x = ne_ref[...] + jnp.where(is0, d0, d1)
    sq = jnp.sum(x * x, axis=1, keepdims=True)
    out_ref[...] = x * lax.rsqrt(jnp.maximum(sq, 1e-12))


BS = 2048


@functools.partial(jax.jit)
def _tc_combine(types2d, ne_g, t0, t1, s1, s2r, w):
    return pl.pallas_call(
        _tc_body,
        grid=(B // BS,),
        in_specs=[
            pl.BlockSpec((BS, 1), lambda i: (i, 0)),
            pl.BlockSpec((BS, EMB), lambda i: (i, 0)),
            pl.BlockSpec((BS, EMB_U), lambda i: (i, 0)),
            pl.BlockSpec((BS, EMB_U), lambda i: (i, 0)),
            pl.BlockSpec((T, EMB_U, DIM_A), lambda i: (0, 0, 0)),
            pl.BlockSpec((T, 1, DIM_A), lambda i: (0, 0, 0)),
            pl.BlockSpec((T, EMB_U, EMB), lambda i: (0, 0, 0)),
        ],
        out_specs=pl.BlockSpec((BS, EMB), lambda i: (i, 0)),
        out_shape=jax.ShapeDtypeStruct((B, EMB), jnp.float32),
    )(types2d, ne_g, t0, t1, s1, s2r, w)


def kernel(node_embeddings, node_type_embeddings, trans_weights,
           trans_weights_s1, trans_weights_s2, train_inputs, train_labels,
           train_types, node_neigh):
    del train_labels
    ntt_flat = node_type_embeddings.reshape(NUM_NODES * T, EMB_U)
    nn_flat = node_neigh.reshape(-1)
    ne_g, t0, t1 = _build_sc_gather()(node_embeddings, ntt_flat, train_inputs,
                                      nn_flat)
    types2d = train_types.reshape(B, 1)
    s2r = jnp.transpose(trans_weights_s2, (0, 2, 1))
    return _tc_combine(types2d, ne_g, t0, t1, trans_weights_s1, s2r,
                       trans_weights)


# speedup vs baseline: 78.0544x; 1.2703x over previous
import functools

import jax
import jax.numpy as jnp
from jax import lax
from jax.experimental import pallas as pl
from jax.experimental.pallas import tpu as pltpu
from jax.experimental.pallas import tpu_sc as plsc

NUM_NODES = 100000
EMB = 64
EMB_U = 16
T = 2
DIM_A = 20
NEIGH = 10
B = 16384

NW = 32
CH = 128
NT = B // CH
TPW = NT // NW
ROWS = CH * T * NEIGH


def _sc_body(ne_hbm, ntt_hbm, ti_hbm, nn_hbm,
             out_ne, out_t0, out_t1,
             idx_v, rows_v, acc0_v, acc1_v, tidx_v, nerows_v,
             sem_g, sem_ne):
    c = lax.axis_index("c")
    s = lax.axis_index("s")
    wid = s * 2 + c
    t_base = wid * TPW

    gather_descs = [None, None]
    ne_descs = [None, None]

    def stage(t, buf):
        pltpu.sync_copy(nn_hbm.at[:, t], idx_v[buf])

        def _fix(j, carry):
            for k in range(NEIGH):
                for i in range(T):
                    sl = (k, i, pl.ds(j * 16, 16))
                    idx_v[buf][sl] = idx_v[buf][sl] * T + i
            return carry

        lax.fori_loop(0, CH // 16, _fix, 0)

        gather_descs[buf] = [
            pltpu.async_copy(
                ntt_hbm.at[idx_v[buf].at[k, i]],
                rows_v[buf].at[pl.ds((k * T + i) * CH, CH)], sem_g[buf])
            for k in range(NEIGH) for i in range(T)
        ]
        pltpu.sync_copy(ti_hbm.at[pl.ds(t * CH, CH)], tidx_v[buf])
        ne_descs[buf] = pltpu.async_copy(
            ne_hbm.at[tidx_v[buf]], nerows_v[buf], sem_ne[buf])

    def consume(t, buf):
        for d in gather_descs[buf]:
            d.wait()

        def _accum(e, carry):
            r0 = rows_v[buf][e]
            r1 = rows_v[buf][CH + e]
            for k in range(1, NEIGH):
                r0 = r0 + rows_v[buf][(k * T) * CH + e]
                r1 = r1 + rows_v[buf][(k * T + 1) * CH + e]
            acc0_v[e] = r0 * (1.0 / NEIGH)
            acc1_v[e] = r1 * (1.0 / NEIGH)
            return carry

        lax.fori_loop(0, CH, _accum, 0)
        pltpu.sync_copy(acc0_v, out_t0.at[pl.ds(t * CH, CH)])
        pltpu.sync_copy(acc1_v, out_t1.at[pl.ds(t * CH, CH)])
        ne_descs[buf].wait()
        pltpu.sync_copy(nerows_v[buf], out_ne.at[pl.ds(t * CH, CH)])

    stage(t_base, 0)
    for q in range(TPW):
        if q + 1 < TPW:
            stage(t_base + q + 1, (q + 1) % 2)
        consume(t_base + q, q % 2)


@functools.cache
def _build_sc_gather():
    return pl.kernel(
        _sc_body,
        out_type=[
            jax.ShapeDtypeStruct((B, EMB), jnp.float32),
            jax.ShapeDtypeStruct((B, EMB_U), jnp.float32),
            jax.ShapeDtypeStruct((B, EMB_U), jnp.float32),
        ],
        mesh=plsc.VectorSubcoreMesh(core_axis_name="c", subcore_axis_name="s"),
        compiler_params=pltpu.CompilerParams(use_tc_tiling_on_sc=False),
        scratch_types=[
            [pltpu.VMEM((NEIGH, T, CH), jnp.int32) for _ in range(2)],
            [pltpu.VMEM((ROWS, EMB_U), jnp.float32) for _ in range(2)],
            pltpu.VMEM((CH, EMB_U), jnp.float32),
            pltpu.VMEM((CH, EMB_U), jnp.float32),
            [pltpu.VMEM((CH,), jnp.int32) for _ in range(2)],
            [pltpu.VMEM((CH, EMB), jnp.float32) for _ in range(2)],
            [pltpu.SemaphoreType.DMA for _ in range(2)],
            [pltpu.SemaphoreType.DMA for _ in range(2)],
        ],
    )


def _tc_body(types_ref, ne_ref, t0_ref, t1_ref, s1_ref, s2_ref, w_ref, out_ref):
    nte0 = t0_ref[...]
    nte1 = t1_ref[...]
    is0 = types_ref[...] == 0
    logit = []
    for tt in range(T):
        s1t = s1_ref[tt]
        s2t = s2_ref[tt]
        h0 = jnp.tanh(jnp.dot(nte0, s1t, preferred_element_type=jnp.float32))
        h1 = jnp.tanh(jnp.dot(nte1, s1t, preferred_element_type=jnp.float32))
        logit.append((jnp.sum(h0 * s2t, axis=1, keepdims=True),
                      jnp.sum(h1 * s2t, axis=1, keepdims=True)))
    l0 = jnp.where(is0, logit[0][0], logit[1][0])
    l1 = jnp.where(is0, logit[0][1], logit[1][1])
    m = jnp.maximum(l0, l1)
    e0 = jnp.exp(l0 - m)
    e1 = jnp.exp(l1 - m)
    inv = 1.0 / (e0 + e1)
    comb = (e0 * inv) * nte0 + (e1 * inv) * nte1
    d0 = jnp.dot(comb, w_ref[0], preferred_element_type=jnp.float32)
    d1 = jnp.dot(comb, w_ref[1], preferred_element_type=jnp.float32)
    x = ne_ref[...] + jnp.where(is0, d0, d1)
    sq = jnp.sum(x * x, axis=1, keepdims=True)
    out_ref[...] = x * lax.rsqrt(jnp.maximum(sq, 1e-12))


BS = 2048


def _tc_combine(types2d, ne_g, t0, t1, s1, s2r, w):
    return pl.pallas_call(
        _tc_body,
        grid=(B // BS,),
        in_specs=[
            pl.BlockSpec((BS, 1), lambda i: (i, 0)),
            pl.BlockSpec((BS, EMB), lambda i: (i, 0)),
            pl.BlockSpec((BS, EMB_U), lambda i: (i, 0)),
            pl.BlockSpec((BS, EMB_U), lambda i: (i, 0)),
            pl.BlockSpec((T, EMB_U, DIM_A), lambda i: (0, 0, 0)),
            pl.BlockSpec((T, 1, DIM_A), lambda i: (0, 0, 0)),
            pl.BlockSpec((T, EMB_U, EMB), lambda i: (0, 0, 0)),
        ],
        out_specs=pl.BlockSpec((BS, EMB), lambda i: (i, 0)),
        out_shape=jax.ShapeDtypeStruct((B, EMB), jnp.float32),
    )(types2d, ne_g, t0, t1, s1, s2r, w)


def kernel(node_embeddings, node_type_embeddings, trans_weights,
           trans_weights_s1, trans_weights_s2, train_inputs, train_labels,
           train_types, node_neigh):
    del train_labels
    ntt_flat = node_type_embeddings.reshape(NUM_NODES * T, EMB_U)
    nn_sc = (node_neigh.transpose(2, 0, 1)
             .reshape(NEIGH, NT, CH, T)
             .transpose(0, 1, 3, 2))
    ne_g, t0, t1 = _build_sc_gather()(node_embeddings, ntt_flat, train_inputs,
                                      nn_sc)
    types2d = train_types.reshape(B, 1)
    s2r = jnp.transpose(trans_weights_s2, (0, 2, 1))
    return _tc_combine(types2d, ne_g, t0, t1, trans_weights_s1, s2r,
                       trans_weights)


# speedup vs baseline: 103.4320x; 1.3251x over previous
import functools

import jax
import jax.numpy as jnp
from jax import lax
from jax.experimental import pallas as pl
from jax.experimental.pallas import tpu as pltpu
from jax.experimental.pallas import tpu_sc as plsc

NUM_NODES = 100000
EMB = 64
EMB_U = 16
T = 2
DIM_A = 20
NEIGH = 10
B = 16384

NW = 32
CH = 64
NCH = B // CH
CPW = NCH // NW
ROWS = CH * T * NEIGH
NT = B // 128


def _sc_body(ne_hbm, ntt_hbm, ti_hbm, nn_hbm,
             out_ne, out_t0, out_t1,
             idx_v, rows_v, acc0_v, acc1_v, tidx_v, nerows_v,
             sem_g, sem_ne):
    c = lax.axis_index("c")
    s = lax.axis_index("s")
    wid = s * 2 + c
    g_base = wid * CPW

    gather_descs = [None, None]
    ne_descs = [None, None]

    def stage(g, buf):
        t = g // 2
        h = g % 2
        pltpu.sync_copy(nn_hbm.at[:, t, :, pl.ds(h * CH, CH)], idx_v[buf])

        def _fix(j, carry):
            for k in range(NEIGH):
                for i in range(T):
                    sl = (k, i, pl.ds(j * 16, 16))
                    idx_v[buf][sl] = idx_v[buf][sl] * 4 + 2
            return carry

        lax.fori_loop(0, CH // 16, _fix, 0)

        gather_descs[buf] = [
            pltpu.async_copy(
                ntt_hbm.at[idx_v[buf].at[k, i]],
                rows_v[buf].at[pl.ds((k * T + i) * CH, CH)], sem_g[buf])
            for k in range(NEIGH) for i in range(T)
        ]
        pltpu.sync_copy(ti_hbm.at[pl.ds(g * CH, CH)], tidx_v[buf])

        def _fix_ti(j, carry):
            sl = (pl.ds(j * 16, 16),)
            tidx_v[buf][sl] = tidx_v[buf][sl] * 2
            return carry

        lax.fori_loop(0, CH // 16, _fix_ti, 0)
        ne_descs[buf] = pltpu.async_copy(
            ne_hbm.at[tidx_v[buf]], nerows_v[buf], sem_ne[buf])

    def consume(g, buf):
        for d in gather_descs[buf]:
            d.wait()

        def _accum(e, carry):
            r0 = rows_v[buf][e, pl.ds(0, 16)]
            r1 = rows_v[buf][CH + e, pl.ds(16, 16)]
            for k in range(1, NEIGH):
                r0 = r0 + rows_v[buf][(k * T) * CH + e, pl.ds(0, 16)]
                r1 = r1 + rows_v[buf][(k * T + 1) * CH + e, pl.ds(16, 16)]
            acc0_v[e] = r0 * (1.0 / NEIGH)
            acc1_v[e] = r1 * (1.0 / NEIGH)
            return carry

        lax.fori_loop(0, CH, _accum, 0)
        pltpu.sync_copy(acc0_v, out_t0.at[pl.ds(g * CH, CH)])
        pltpu.sync_copy(acc1_v, out_t1.at[pl.ds(g * CH, CH)])
        ne_descs[buf].wait()
        pltpu.sync_copy(nerows_v[buf], out_ne.at[pl.ds(g * CH, CH)])

    stage(g_base, 0)
    for q in range(CPW):
        if q + 1 < CPW:
            stage(g_base + q + 1, (q + 1) % 2)
        consume(g_base + q, q % 2)


@functools.cache
def _build_sc_gather():
    return pl.kernel(
        _sc_body,
        out_type=[
            jax.ShapeDtypeStruct((B, EMB), jnp.float32),
            jax.ShapeDtypeStruct((B, EMB_U), jnp.float32),
            jax.ShapeDtypeStruct((B, EMB_U), jnp.float32),
        ],
        mesh=plsc.VectorSubcoreMesh(core_axis_name="c", subcore_axis_name="s"),
        compiler_params=pltpu.CompilerParams(use_tc_tiling_on_sc=False),
        scratch_types=[
            [pltpu.VMEM((NEIGH, T, CH), jnp.int32) for _ in range(2)],
            [pltpu.VMEM((ROWS, T * EMB_U), jnp.float32) for _ in range(2)],
            pltpu.VMEM((CH, EMB_U), jnp.float32),
            pltpu.VMEM((CH, EMB_U), jnp.float32),
            [pltpu.VMEM((CH,), jnp.int32) for _ in range(2)],
            [pltpu.VMEM((CH, EMB), jnp.float32) for _ in range(2)],
            [pltpu.SemaphoreType.DMA for _ in range(2)],
            [pltpu.SemaphoreType.DMA for _ in range(2)],
        ],
    )


VB = 2048
NVB = -(-NUM_NODES // VB)


def _tp_body(xu_ref, xe_ref, one_ref, ontt_ref):
    row = jnp.concatenate(
        [xe_ref[...].T, xu_ref[...].T,
         jnp.zeros((VB, 32), jnp.float32)], axis=1)
    one_ref[...] = row
    ontt_ref[...] = row


def _tp_tables(ntt_u, ne_u):
    return pl.pallas_call(
        _tp_body,
        grid=(NVB,),
        in_specs=[
            pl.BlockSpec((T * EMB_U, VB), lambda j: (0, j)),
            pl.BlockSpec((EMB, VB), lambda j: (0, j)),
        ],
        out_specs=[
            pl.BlockSpec((VB, 128), lambda j: (j, 0)),
            pl.BlockSpec((VB, 128), lambda j: (j, 0)),
        ],
        out_shape=[
            jax.ShapeDtypeStruct((NUM_NODES, 128), jnp.float32),
            jax.ShapeDtypeStruct((NUM_NODES, 128), jnp.float32),
        ],
    )(ntt_u, ne_u)


def _tc_body(types_ref, ne_ref, t0_ref, t1_ref, s1_ref, s2_ref, w_ref, out_ref):
    nte0 = t0_ref[...]
    nte1 = t1_ref[...]
    is0 = types_ref[...] == 0
    logit = []
    for tt in range(T):
        s1t = s1_ref[tt]
        s2t = s2_ref[tt]
        h0 = jnp.tanh(jnp.dot(nte0, s1t, preferred_element_type=jnp.float32))
        h1 = jnp.tanh(jnp.dot(nte1, s1t, preferred_element_type=jnp.float32))
        logit.append((jnp.sum(h0 * s2t, axis=1, keepdims=True),
                      jnp.sum(h1 * s2t, axis=1, keepdims=True)))
    l0 = jnp.where(is0, logit[0][0], logit[1][0])
    l1 = jnp.where(is0, logit[0][1], logit[1][1])
    m = jnp.maximum(l0, l1)
    e0 = jnp.exp(l0 - m)
    e1 = jnp.exp(l1 - m)
    inv = 1.0 / (e0 + e1)
    comb = (e0 * inv) * nte0 + (e1 * inv) * nte1
    d0 = jnp.dot(comb, w_ref[0], preferred_element_type=jnp.float32)
    d1 = jnp.dot(comb, w_ref[1], preferred_element_type=jnp.float32)
    x = ne_ref[...] + jnp.where(is0, d0, d1)
    sq = jnp.sum(x * x, axis=1, keepdims=True)
    out_ref[...] = x * lax.rsqrt(jnp.maximum(sq, 1e-12))


BS = 2048


def _tc_combine(types2d, ne_g, t0, t1, s1, s2r, w):
    return pl.pallas_call(
        _tc_body,
        grid=(B // BS,),
        in_specs=[
            pl.BlockSpec((BS, 1), lambda i: (i, 0)),
            pl.BlockSpec((BS, EMB), lambda i: (i, 0)),
            pl.BlockSpec((BS, EMB_U), lambda i: (i, 0)),
            pl.BlockSpec((BS, EMB_U), lambda i: (i, 0)),
            pl.BlockSpec((T, EMB_U, DIM_A), lambda i: (0, 0, 0)),
            pl.BlockSpec((T, 1, DIM_A), lambda i: (0, 0, 0)),
            pl.BlockSpec((T, EMB_U, EMB), lambda i: (0, 0, 0)),
        ],
        out_specs=pl.BlockSpec((BS, EMB), lambda i: (i, 0)),
        out_shape=jax.ShapeDtypeStruct((B, EMB), jnp.float32),
    )(types2d, ne_g, t0, t1, s1, s2r, w)


def kernel(node_embeddings, node_type_embeddings, trans_weights,
           trans_weights_s1, trans_weights_s2, train_inputs, train_labels,
           train_types, node_neigh):
    del train_labels
    ntt_u = node_type_embeddings.transpose(1, 2, 0).reshape(T * EMB_U,
                                                            NUM_NODES)
    ne_u = jnp.transpose(node_embeddings)
    ne_pack, ntt_pack = _tp_tables(ntt_u, ne_u)
    ne_tab = ne_pack.reshape(NUM_NODES * 2, EMB)
    ntt_tab = ntt_pack.reshape(NUM_NODES * 4, T * EMB_U)
    nn_sc = (node_neigh.transpose(2, 0, 1)
             .reshape(NEIGH, NT, 128, T)
             .transpose(0, 1, 3, 2))
    ne_g, t0, t1 = _build_sc_gather()(ne_tab, ntt_tab, train_inputs, nn_sc)
    types2d = train_types.reshape(B, 1)
    s2r = jnp.transpose(trans_weights_s2, (0, 2, 1))
    return _tc_combine(types2d, ne_g, t0, t1, trans_weights_s1, s2r,
                       trans_weights)


# speedup vs baseline: 103.5030x; 1.0007x over previous
import functools

import jax
import jax.numpy as jnp
from jax import lax
from jax.experimental import pallas as pl
from jax.experimental.pallas import tpu as pltpu
from jax.experimental.pallas import tpu_sc as plsc

NUM_NODES = 100000
EMB = 64
EMB_U = 16
T = 2
DIM_A = 20
NEIGH = 10
B = 16384

NW = 32
CH = 64
NCH = B // CH
CPW = NCH // NW
ROWS = CH * T * NEIGH
NT = B // 128


def _sc_body(tab_hbm, ti_hbm, nn_hbm,
             out_ne0, out_ne1, out_t0, out_t1,
             idx_v, rows_v, acc0_v, acc1_v, tidx_v, nerows_v,
             sem_g, sem_ne):
    c = lax.axis_index("c")
    s = lax.axis_index("s")
    wid = s * 2 + c
    g_base = wid * CPW

    gather_descs = [None, None]
    ne_descs = [None, None]

    def stage(g, buf):
        t = g // 2
        h = g % 2
        pltpu.sync_copy(nn_hbm.at[:, t, :, pl.ds(h * CH, CH)], idx_v[buf])

        def _fix(j, carry):
            for k in range(NEIGH):
                for i in range(T):
                    sl = (k, i, pl.ds(j * 16, 16))
                    idx_v[buf][sl] = idx_v[buf][sl] * 4 + 2
            return carry

        lax.fori_loop(0, CH // 16, _fix, 0)

        gather_descs[buf] = [
            pltpu.async_copy(
                tab_hbm.at[idx_v[buf].at[k, i]],
                rows_v[buf].at[pl.ds((k * T + i) * CH, CH)], sem_g[buf])
            for k in range(NEIGH) for i in range(T)
        ]
        pltpu.sync_copy(ti_hbm.at[pl.ds(g * CH, CH)],
                        tidx_v[buf].at[pl.ds(0, CH)])

        def _fix_ti(j, carry):
            lo = (pl.ds(j * 16, 16),)
            hi = (pl.ds(CH + j * 16, 16),)
            tidx_v[buf][hi] = tidx_v[buf][lo] * 4 + 1
            tidx_v[buf][lo] = tidx_v[buf][lo] * 4
            return carry

        lax.fori_loop(0, CH // 16, _fix_ti, 0)
        ne_descs[buf] = [
            pltpu.async_copy(
                tab_hbm.at[tidx_v[buf].at[pl.ds(h * CH, CH)]],
                nerows_v[buf].at[pl.ds(h * CH, CH)], sem_ne[buf])
            for h in range(2)
        ]

    def consume(g, buf):
        for d in gather_descs[buf]:
            d.wait()

        def _accum(e, carry):
            r0 = rows_v[buf][e, pl.ds(0, 16)]
            r1 = rows_v[buf][CH + e, pl.ds(16, 16)]
            for k in range(1, NEIGH):
                r0 = r0 + rows_v[buf][(k * T) * CH + e, pl.ds(0, 16)]
                r1 = r1 + rows_v[buf][(k * T + 1) * CH + e, pl.ds(16, 16)]
            acc0_v[e] = r0 * (1.0 / NEIGH)
            acc1_v[e] = r1 * (1.0 / NEIGH)
            return carry

        lax.fori_loop(0, CH, _accum, 0)
        pltpu.sync_copy(acc0_v, out_t0.at[pl.ds(g * CH, CH)])
        pltpu.sync_copy(acc1_v, out_t1.at[pl.ds(g * CH, CH)])
        for d in ne_descs[buf]:
            d.wait()
        pltpu.sync_copy(nerows_v[buf].at[pl.ds(0, CH)],
                        out_ne0.at[pl.ds(g * CH, CH)])
        pltpu.sync_copy(nerows_v[buf].at[pl.ds(CH, CH)],
                        out_ne1.at[pl.ds(g * CH, CH)])

    stage(g_base, 0)
    for q in range(CPW):
        if q + 1 < CPW:
            stage(g_base + q + 1, (q + 1) % 2)
        consume(g_base + q, q % 2)


@functools.cache
def _build_sc_gather():
    return pl.kernel(
        _sc_body,
        out_type=[
            jax.ShapeDtypeStruct((B, 32), jnp.float32),
            jax.ShapeDtypeStruct((B, 32), jnp.float32),
            jax.ShapeDtypeStruct((B, EMB_U), jnp.float32),
            jax.ShapeDtypeStruct((B, EMB_U), jnp.float32),
        ],
        mesh=plsc.VectorSubcoreMesh(core_axis_name="c", subcore_axis_name="s"),
        compiler_params=pltpu.CompilerParams(use_tc_tiling_on_sc=False),
        scratch_types=[
            [pltpu.VMEM((NEIGH, T, CH), jnp.int32) for _ in range(2)],
            [pltpu.VMEM((ROWS, T * EMB_U), jnp.float32) for _ in range(2)],
            pltpu.VMEM((CH, EMB_U), jnp.float32),
            pltpu.VMEM((CH, EMB_U), jnp.float32),
            [pltpu.VMEM((2 * CH,), jnp.int32) for _ in range(2)],
            [pltpu.VMEM((2 * CH, 32), jnp.float32) for _ in range(2)],
            [pltpu.SemaphoreType.DMA for _ in range(2)],
            [pltpu.SemaphoreType.DMA for _ in range(2)],
        ],
    )


VB = 2048
NVB = -(-NUM_NODES // VB)


def _tp_body(xu_ref, xe_ref, o_ref):
    o_ref[...] = jnp.concatenate(
        [xe_ref[...].T, xu_ref[...].T,
         jnp.zeros((VB, 32), jnp.float32)], axis=1)


def _tp_tables(ntt_u, ne_u):
    return pl.pallas_call(
        _tp_body,
        grid=(NVB,),
        in_specs=[
            pl.BlockSpec((T * EMB_U, VB), lambda j: (0, j)),
            pl.BlockSpec((EMB, VB), lambda j: (0, j)),
        ],
        out_specs=pl.BlockSpec((VB, 128), lambda j: (j, 0)),
        out_shape=jax.ShapeDtypeStruct((NUM_NODES, 128), jnp.float32),
    )(ntt_u, ne_u)


def _tc_body(types_ref, ne0_ref, ne1_ref, t0_ref, t1_ref, s1_ref, s2_ref,
             w_ref, out_ref):
    nte0 = t0_ref[...]
    nte1 = t1_ref[...]
    is0 = types_ref[...] == 0
    logit = []
    for tt in range(T):
        s1t = s1_ref[tt]
        s2t = s2_ref[tt]
        h0 = jnp.tanh(jnp.dot(nte0, s1t, preferred_element_type=jnp.float32))
        h1 = jnp.tanh(jnp.dot(nte1, s1t, preferred_element_type=jnp.float32))
        logit.append((jnp.sum(h0 * s2t, axis=1, keepdims=True),
                      jnp.sum(h1 * s2t, axis=1, keepdims=True)))
    l0 = jnp.where(is0, logit[0][0], logit[1][0])
    l1 = jnp.where(is0, logit[0][1], logit[1][1])
    m = jnp.maximum(l0, l1)
    e0 = jnp.exp(l0 - m)
    e1 = jnp.exp(l1 - m)
    inv = 1.0 / (e0 + e1)
    comb = (e0 * inv) * nte0 + (e1 * inv) * nte1
    d0 = jnp.dot(comb, w_ref[0], preferred_element_type=jnp.float32)
    d1 = jnp.dot(comb, w_ref[1], preferred_element_type=jnp.float32)
    ne = jnp.concatenate([ne0_ref[...], ne1_ref[...]], axis=1)
    x = ne + jnp.where(is0, d0, d1)
    sq = jnp.sum(x * x, axis=1, keepdims=True)
    out_ref[...] = x * lax.rsqrt(jnp.maximum(sq, 1e-12))


BS = 2048


def _tc_combine(types2d, ne0, ne1, t0, t1, s1, s2r, w):
    return pl.pallas_call(
        _tc_body,
        grid=(B // BS,),
        in_specs=[
            pl.BlockSpec((BS, 1), lambda i: (i, 0)),
            pl.BlockSpec((BS, 32), lambda i: (i, 0)),
            pl.BlockSpec((BS, 32), lambda i: (i, 0)),
            pl.BlockSpec((BS, EMB_U), lambda i: (i, 0)),
            pl.BlockSpec((BS, EMB_U), lambda i: (i, 0)),
            pl.BlockSpec((T, EMB_U, DIM_A), lambda i: (0, 0, 0)),
            pl.BlockSpec((T, 1, DIM_A), lambda i: (0, 0, 0)),
            pl.BlockSpec((T, EMB_U, EMB), lambda i: (0, 0, 0)),
        ],
        out_specs=pl.BlockSpec((BS, EMB), lambda i: (i, 0)),
        out_shape=jax.ShapeDtypeStruct((B, EMB), jnp.float32),
    )(types2d, ne0, ne1, t0, t1, s1, s2r, w)


def kernel(node_embeddings, node_type_embeddings, trans_weights,
           trans_weights_s1, trans_weights_s2, train_inputs, train_labels,
           train_types, node_neigh):
    del train_labels
    ntt_u = node_type_embeddings.transpose(1, 2, 0).reshape(T * EMB_U,
                                                            NUM_NODES)
    ne_u = jnp.transpose(node_embeddings)
    tab = _tp_tables(ntt_u, ne_u).reshape(NUM_NODES * 4, T * EMB_U)
    nn_sc = (node_neigh.transpose(2, 0, 1)
             .reshape(NEIGH, NT, 128, T)
             .transpose(0, 1, 3, 2))
    ne0, ne1, t0, t1 = _build_sc_gather()(tab, train_inputs, nn_sc)
    types2d = train_types.reshape(B, 1)
    s2r = jnp.transpose(trans_weights_s2, (0, 2, 1))
    return _tc_combine(types2d, ne0, ne1, t0, t1, trans_weights_s1, s2r,
                       trans_weights)
